# baseline (speedup 1.0000x reference)
import functools

import jax
import jax.numpy as jnp
from jax.experimental import pallas as pl

_BETA = 0.25


def _vq_block_kernel(flat_ref, cb_ref, qst_ref, loss_ref, *, n_blocks, inv_count):
    i = pl.program_id(0)
    flat = flat_ref[...]
    cb = cb_ref[...]
    bn = flat.shape[0]
    k = cb.shape[0]

    mm = jax.lax.dot_general(
        flat, cb,
        dimension_numbers=(((1,), (1,)), ((), ())),
        preferred_element_type=jnp.float32,
    )
    zsq = jnp.sum(flat * flat, axis=1, keepdims=True)
    c2 = jnp.sum(cb * cb, axis=1)[None, :]
    dist = (zsq + c2) - 2.0 * mm

    minval = jnp.min(dist, axis=1, keepdims=True)
    lane = jax.lax.broadcasted_iota(jnp.int32, (bn, k), 1)
    idx = jnp.min(jnp.where(dist == minval, lane, k), axis=1, keepdims=True)

    onehot = (lane == idx).astype(jnp.float32)
    q = jax.lax.dot_general(
        onehot, cb,
        dimension_numbers=(((1,), (0,)), ((), ())),
        preferred_element_type=jnp.float32,
    )

    qst_ref[...] = flat + (q - flat)

    diff = q - flat
    part = jnp.sum(diff * diff).reshape(1, 1)

    @pl.when(i == 0)
    def _init():
        loss_ref[...] = jnp.zeros((1, 1), jnp.float32)

    loss_ref[...] += part

    @pl.when(i == n_blocks - 1)
    def _finish():
        loss_ref[...] = loss_ref[...] * ((1.0 + _BETA) * inv_count)


@jax.jit
def kernel(data, codebook):
    orig_shape = data.shape
    d = data.shape[-1]
    flat = data.reshape(-1, d)
    n = flat.shape[0]
    k = codebook.shape[0]

    bn = 256
    n_blocks = n // bn
    inv_count = 1.0 / float(data.size)

    qst, loss = pl.pallas_call(
        functools.partial(_vq_block_kernel, n_blocks=n_blocks, inv_count=inv_count),
        grid=(n_blocks,),
        in_specs=[
            pl.BlockSpec((bn, d), lambda i: (i, 0)),
            pl.BlockSpec((k, d), lambda i: (0, 0)),
        ],
        out_specs=[
            pl.BlockSpec((bn, d), lambda i: (i, 0)),
            pl.BlockSpec((1, 1), lambda i: (0, 0)),
        ],
        out_shape=[
            jax.ShapeDtypeStruct((n, d), jnp.float32),
            jax.ShapeDtypeStruct((1, 1), jnp.float32),
        ],
    )(flat, codebook)

    return qst.reshape(orig_shape), loss[0, 0]
